# baseline (speedup 1.0000x reference)
import functools

import jax
import jax.numpy as jnp
from jax import lax
from jax.experimental import pallas as pl
from jax.experimental.pallas import tpu as pltpu
from jax.experimental.pallas import tpu_sc as plsc

_VOCAB = 100000
_SEQ = 200
_DIM = 64
_BATCH = 4096

_NC = 2
_NS = 16
_NW = _NC * _NS
_TOTAL = _BATCH * _SEQ
_PER_W = _TOTAL // _NW
_CH = 128
_NCHUNK = _PER_W // _CH
_LANES = 16
_NBUF = 4
_PD = 2


def _sc_embed(idx_flat, token_table, pos_table):
    mesh = plsc.VectorSubcoreMesh(core_axis_name="c", subcore_axis_name="s")

    @functools.partial(
        pl.kernel,
        out_type=jax.ShapeDtypeStruct((_TOTAL, _DIM), jnp.float32),
        mesh=mesh,
        compiler_params=pltpu.CompilerParams(use_tc_tiling_on_sc=False),
        scratch_types=[
            pltpu.VMEM((2 * _SEQ, _DIM), jnp.float32),
            pltpu.VMEM((_PER_W,), jnp.int32),
            [pltpu.VMEM((_CH, _DIM), jnp.float32)] * _NBUF,
            [pltpu.SemaphoreType.DMA] * _NBUF,
            [pltpu.SemaphoreType.DMA] * _NBUF,
        ],
    )
    def k(idx_hbm, tok_hbm, pos_hbm, out_hbm, pos2_v, idx_v, rows, gsem, osem):
        wid = lax.axis_index("s") * _NC + lax.axis_index("c")
        base = wid * _PER_W
        pltpu.sync_copy(idx_hbm.at[pl.ds(base, _PER_W)], idx_v)
        pltpu.sync_copy(pos_hbm, pos2_v.at[pl.ds(0, _SEQ)])
        pltpu.sync_copy(pos_hbm, pos2_v.at[pl.ds(_SEQ, _SEQ)])

        def gather(j, b):
            return pltpu.make_async_copy(
                tok_hbm.at[idx_v.at[pl.ds(j * _CH, _CH)]], rows[b], gsem[b])

        def store(j, b):
            return pltpu.make_async_copy(
                rows[b], out_hbm.at[pl.ds(base + j * _CH, _CH)], osem[b])

        for b in range(_PD):
            gather(b, b).start()

        @pl.loop(0, _NCHUNK, step=_NBUF)
        def _chunks(i0):
            for b in range(_NBUF):
                i = i0 + b
                j = i + _PD
                bj = (b + _PD) % _NBUF

                @pl.when(j < _NCHUNK)
                def _():
                    @pl.when(j >= _NBUF)
                    def _():
                        store(0, bj).wait()

                    gather(j, bj).start()

                gather(i, b).wait()
                p0 = lax.rem(i * _CH, _SEQ)

                @pl.loop(0, _CH, unroll=4)
                def _row(r):
                    pr = p0 + r
                    for c in range(0, _DIM, _LANES):
                        rows[b][r, pl.ds(c, _LANES)] = (
                            rows[b][r, pl.ds(c, _LANES)]
                            + pos2_v[pr, pl.ds(c, _LANES)]
                        )

                store(i, b).start()

        for b in range(_NBUF):
            store(0, b).wait()

    return k(idx_flat, token_table, pos_table)


def kernel(inputs, token_table, pos_table):
    idx_flat = jnp.reshape(inputs, (-1,)).astype(jnp.int32)
    out = _sc_embed(idx_flat, token_table, pos_table)
    return out.reshape(_BATCH, _SEQ, _DIM)


# speedup vs baseline: 1.1212x; 1.1212x over previous
import functools

import jax
import jax.numpy as jnp
from jax import lax
from jax.experimental import pallas as pl
from jax.experimental.pallas import tpu as pltpu
from jax.experimental.pallas import tpu_sc as plsc

_VOCAB = 100000
_SEQ = 200
_DIM = 64
_BATCH = 4096

_NC = 2
_NS = 16
_NW = _NC * _NS
_TOTAL = _BATCH * _SEQ
_PER_W = _TOTAL // _NW
_CH = 128
_NCHUNK = _PER_W // _CH
_LANES = 16
_NBUF = 4
_PD = 2


def _sc_embed(idx_flat, token_table, pos_table):
    mesh = plsc.VectorSubcoreMesh(core_axis_name="c", subcore_axis_name="s")

    @functools.partial(
        pl.kernel,
        out_type=jax.ShapeDtypeStruct((_TOTAL, _DIM), jnp.float32),
        mesh=mesh,
        compiler_params=pltpu.CompilerParams(use_tc_tiling_on_sc=False),
        scratch_types=[
            pltpu.VMEM((2 * _SEQ, _DIM), jnp.float32),
            pltpu.VMEM((_PER_W,), jnp.int32),
            [pltpu.VMEM((_CH, _DIM), jnp.float32)] * _NBUF,
            [pltpu.SemaphoreType.DMA] * _NBUF,
            [pltpu.SemaphoreType.DMA] * _NBUF,
        ],
    )
    def k(idx_hbm, tok_hbm, pos_hbm, out_hbm, pos2_v, idx_v, rows, gsem, osem):
        wid = lax.axis_index("s") * _NC + lax.axis_index("c")
        base = wid * _PER_W
        pltpu.sync_copy(idx_hbm.at[pl.ds(base, _PER_W)], idx_v)
        pltpu.sync_copy(pos_hbm, pos2_v.at[pl.ds(0, _SEQ)])
        pltpu.sync_copy(pos_hbm, pos2_v.at[pl.ds(_SEQ, _SEQ)])

        def gather_add(j, b):
            pj = lax.rem(j * _CH, _SEQ)

            @pl.loop(0, _CH, unroll=4)
            def _fill(r):
                pr = pj + r
                for c in range(0, _DIM, _LANES):
                    rows[b][r, pl.ds(c, _LANES)] = pos2_v[pr, pl.ds(c, _LANES)]
            pltpu.async_copy(
                tok_hbm.at[idx_v.at[pl.ds(j * _CH, _CH)]], rows[b], gsem[b],
                add=True)

        def gather_wait(b):
            return pltpu.make_async_copy(
                tok_hbm.at[idx_v.at[pl.ds(0, _CH)]], rows[b], gsem[b])

        def store(j, b):
            return pltpu.make_async_copy(
                rows[b], out_hbm.at[pl.ds(base + j * _CH, _CH)], osem[b])

        for b in range(_PD):
            gather_add(b, b)

        @pl.loop(0, _NCHUNK, step=_NBUF)
        def _chunks(i0):
            for b in range(_NBUF):
                i = i0 + b
                j = i + _PD
                bj = (b + _PD) % _NBUF

                @pl.when(j < _NCHUNK)
                def _():
                    @pl.when(j >= _NBUF)
                    def _():
                        store(0, bj).wait()

                    gather_add(j, bj)

                gather_wait(b).wait()
                store(i, b).start()

        for b in range(_NBUF):
            store(0, b).wait()

    return k(idx_flat, token_table, pos_table)


def kernel(inputs, token_table, pos_table):
    idx_flat = jnp.reshape(inputs, (-1,)).astype(jnp.int32)
    out = _sc_embed(idx_flat, token_table, pos_table)
    return out.reshape(_BATCH, _SEQ, _DIM)


# speedup vs baseline: 1.4636x; 1.3054x over previous
import functools

import jax
import jax.numpy as jnp
from jax import lax
from jax.experimental import pallas as pl
from jax.experimental.pallas import tpu as pltpu
from jax.experimental.pallas import tpu_sc as plsc

_VOCAB = 100000
_SEQ = 200
_DIM = 64
_BATCH = 4096

_NC = 2
_NS = 16
_NW = _NC * _NS
_TOTAL = _BATCH * _SEQ
_PER_W = _TOTAL // _NW
_CH = 128
_NCHUNK = _PER_W // _CH
_NBUF = 5
_FD = 3
_PD = 2


def _sc_embed(idx_flat, token_table, pos_table):
    mesh = plsc.VectorSubcoreMesh(core_axis_name="c", subcore_axis_name="s")

    @functools.partial(
        pl.kernel,
        out_type=jax.ShapeDtypeStruct((_TOTAL, _DIM), jnp.float32),
        mesh=mesh,
        compiler_params=pltpu.CompilerParams(use_tc_tiling_on_sc=False),
        scratch_types=[
            pltpu.VMEM_SHARED((2 * _SEQ, _DIM), jnp.float32),
            pltpu.VMEM((_PER_W,), jnp.int32),
            [pltpu.VMEM((_CH, _DIM), jnp.float32)] * _NBUF,
            [pltpu.SemaphoreType.DMA] * _NBUF,
            [pltpu.SemaphoreType.DMA] * _NBUF,
            [pltpu.SemaphoreType.DMA] * _NBUF,
        ],
    )
    def k(idx_hbm, tok_hbm, pos_hbm, out_hbm,
          pos2_s, idx_v, rows, fsem, gsem, osem):
        wid = lax.axis_index("s") * _NC + lax.axis_index("c")
        base = wid * _PER_W
        pltpu.sync_copy(idx_hbm.at[pl.ds(base, _PER_W)], idx_v)

        @pl.when(lax.axis_index("s") == 0)
        def _():
            pltpu.sync_copy(pos_hbm, pos2_s.at[pl.ds(0, _SEQ)])
            pltpu.sync_copy(pos_hbm, pos2_s.at[pl.ds(_SEQ, _SEQ)])

        plsc.subcore_barrier()

        def fill(j, b):
            pj = lax.rem(j * _CH, _SEQ)
            pltpu.async_copy(pos2_s.at[pl.ds(pj, _CH)], rows[b], fsem[b])

        def fill_wait(b):
            pltpu.make_async_copy(
                pos2_s.at[pl.ds(0, _CH)], rows[b], fsem[b]).wait()

        def gather_add(j, b):
            pltpu.async_copy(
                tok_hbm.at[idx_v.at[pl.ds(j * _CH, _CH)]], rows[b], gsem[b],
                add=True)

        def gather_wait(b):
            pltpu.make_async_copy(
                tok_hbm.at[idx_v.at[pl.ds(0, _CH)]], rows[b], gsem[b]).wait()

        def store(j, b):
            return pltpu.make_async_copy(
                rows[b], out_hbm.at[pl.ds(base + j * _CH, _CH)], osem[b])

        for b in range(_FD):
            fill(b, b)
        for b in range(_PD):
            fill_wait(b)
            gather_add(b, b)

        @pl.loop(0, _NCHUNK, step=_NBUF)
        def _chunks(i0):
            for b in range(_NBUF):
                i = i0 + b
                jf = i + _FD
                bf = (b + _FD) % _NBUF

                @pl.when(jf < _NCHUNK)
                def _():
                    @pl.when(jf >= _NBUF)
                    def _():
                        store(0, bf).wait()

                    fill(jf, bf)

                jg = i + _PD
                bg = (b + _PD) % _NBUF

                @pl.when(jg < _NCHUNK)
                def _():
                    fill_wait(bg)
                    gather_add(jg, bg)

                gather_wait(b)
                store(i, b).start()

        for b in range(_NBUF):
            store(0, b).wait()

    return k(idx_flat, token_table, pos_table)


def kernel(inputs, token_table, pos_table):
    idx_flat = jnp.reshape(inputs, (-1,)).astype(jnp.int32)
    out = _sc_embed(idx_flat, token_table, pos_table)
    return out.reshape(_BATCH, _SEQ, _DIM)


# speedup vs baseline: 1.4669x; 1.0022x over previous
import functools

import jax
import jax.numpy as jnp
from jax import lax
from jax.experimental import pallas as pl
from jax.experimental.pallas import tpu as pltpu
from jax.experimental.pallas import tpu_sc as plsc

_VOCAB = 100000
_SEQ = 200
_DIM = 64
_BATCH = 4096

_NC = 2
_NS = 16
_NW = _NC * _NS
_TOTAL = _BATCH * _SEQ
_PER_W = _TOTAL // _NW
_CH = 128
_NCHUNK = _PER_W // _CH
_NBUF = 8
_FD = 5
_PD = 4


def _sc_embed(idx_flat, token_table, pos_table):
    mesh = plsc.VectorSubcoreMesh(core_axis_name="c", subcore_axis_name="s")

    @functools.partial(
        pl.kernel,
        out_type=jax.ShapeDtypeStruct((_TOTAL, _DIM), jnp.float32),
        mesh=mesh,
        compiler_params=pltpu.CompilerParams(use_tc_tiling_on_sc=False),
        scratch_types=[
            pltpu.VMEM_SHARED((2 * _SEQ, _DIM), jnp.float32),
            pltpu.VMEM((_PER_W,), jnp.int32),
            [pltpu.VMEM((_CH, _DIM), jnp.float32)] * _NBUF,
            [pltpu.SemaphoreType.DMA] * _NBUF,
            [pltpu.SemaphoreType.DMA] * _NBUF,
            [pltpu.SemaphoreType.DMA] * _NBUF,
        ],
    )
    def k(idx_hbm, tok_hbm, pos_hbm, out_hbm,
          pos2_s, idx_v, rows, fsem, gsem, osem):
        wid = lax.axis_index("s") * _NC + lax.axis_index("c")
        base = wid * _PER_W
        pltpu.sync_copy(idx_hbm.at[pl.ds(base, _PER_W)], idx_v)

        @pl.when(lax.axis_index("s") == 0)
        def _():
            pltpu.sync_copy(pos_hbm, pos2_s.at[pl.ds(0, _SEQ)])
            pltpu.sync_copy(pos_hbm, pos2_s.at[pl.ds(_SEQ, _SEQ)])

        plsc.subcore_barrier()

        def fill(j, b):
            pj = lax.rem(j * _CH, _SEQ)
            pltpu.async_copy(pos2_s.at[pl.ds(pj, _CH)], rows[b], fsem[b])

        def fill_wait(b):
            pltpu.make_async_copy(
                pos2_s.at[pl.ds(0, _CH)], rows[b], fsem[b]).wait()

        def gather_add(j, b):
            pltpu.async_copy(
                tok_hbm.at[idx_v.at[pl.ds(j * _CH, _CH)]], rows[b], gsem[b],
                add=True)

        def gather_wait(b):
            pltpu.make_async_copy(
                tok_hbm.at[idx_v.at[pl.ds(0, _CH)]], rows[b], gsem[b]).wait()

        def store(j, b):
            return pltpu.make_async_copy(
                rows[b], out_hbm.at[pl.ds(base + j * _CH, _CH)], osem[b])

        for b in range(_FD):
            fill(b, b)
        for b in range(_PD):
            fill_wait(b)
            gather_add(b, b)

        @pl.loop(0, _NCHUNK, step=_NBUF)
        def _chunks(i0):
            for b in range(_NBUF):
                i = i0 + b
                jf = i + _FD
                bf = (b + _FD) % _NBUF

                @pl.when(jf < _NCHUNK)
                def _():
                    @pl.when(jf >= _NBUF)
                    def _():
                        store(0, bf).wait()

                    fill(jf, bf)

                jg = i + _PD
                bg = (b + _PD) % _NBUF

                @pl.when(jg < _NCHUNK)
                def _():
                    fill_wait(bg)
                    gather_add(jg, bg)

                gather_wait(b)
                store(i, b).start()

        for b in range(_NBUF):
            store(0, b).wait()

    return k(idx_flat, token_table, pos_table)


def kernel(inputs, token_table, pos_table):
    idx_flat = jnp.reshape(inputs, (-1,)).astype(jnp.int32)
    out = _sc_embed(idx_flat, token_table, pos_table)
    return out.reshape(_BATCH, _SEQ, _DIM)
